# baseline (speedup 1.0000x reference)
import jax
import jax.numpy as jnp
from jax.experimental import pallas as pl
from jax.experimental.pallas import tpu as pltpu

_B, _DIN, _D, _K, _P = 4096, 1024, 510, 128, 332
_DP = 512
_PP = 384
_TB = 256
_G = _B // _TB


def _fused(pn_ref, xt_ref, xp_ref, wm_ref, bm_ref, cb_ref, cbt_ref, cbn_ref,
           pt_ref, wd_ref, bd_ref, out_ref, z_ref, zq_ref):
    wm = wm_ref[...]
    bm = bm_ref[...]
    hi = jax.lax.Precision.HIGHEST
    zt = jnp.dot(xt_ref[...], wm, preferred_element_type=jnp.float32) + bm
    zp = jnp.dot(xp_ref[...], wm, preferred_element_type=jnp.float32) + bm

    cb = cb_ref[...]
    cbt = cbt_ref[...]
    cn = cbn_ref[...]
    st = cn - 2.0 * jnp.dot(
        zt, cbt, preferred_element_type=jnp.float32, precision=hi)
    sp = cn - 2.0 * jnp.dot(
        zp, cbt, preferred_element_type=jnp.float32, precision=hi)
    kt = jnp.argmin(st, axis=1)
    kp = jnp.argmin(sp, axis=1)
    iota_k = jax.lax.broadcasted_iota(jnp.int32, (_TB, _K), 1)
    oh_t = (iota_k == kt[:, None]).astype(jnp.float32)
    oh_p = (iota_k == kp[:, None]).astype(jnp.float32)
    zqt = jnp.dot(oh_t, cb, preferred_element_type=jnp.float32, precision=hi)
    zqp = jnp.dot(oh_p, cb, preferred_element_type=jnp.float32, precision=hi)

    pn = pn_ref[0, 0, :]
    iota_p = jax.lax.broadcasted_iota(jnp.int32, (_TB, _PP), 1)
    oh_pos = (iota_p == pn[:, None]).astype(jnp.float32)
    pos = jnp.dot(oh_pos, pt_ref[...], preferred_element_type=jnp.float32,
                  precision=hi)

    acc = zqt + zqp + pos
    logits = jnp.dot(acc, wd_ref[...],
                     preferred_element_type=jnp.float32) + bd_ref[...]
    out_ref[...] = jax.nn.sigmoid(logits)
    z_ref[...] = zt[:, :_D]
    zq_ref[...] = zqt[:, :_D]


def kernel(train_data, pre_phrase, position_number, W_mean, b_mean, W_var,
           b_var, W_dec, b_dec, codebook, phrase_table):
    del W_var, b_var

    xt16 = train_data.astype(jnp.bfloat16)
    xp16 = pre_phrase.astype(jnp.bfloat16)
    wm = jnp.pad(W_mean.astype(jnp.bfloat16), ((0, 0), (0, _DP - _D)))
    bm = jnp.pad(b_mean, (0, _DP - _D)).reshape(1, _DP)
    cb = jnp.pad(codebook, ((0, 0), (0, _DP - _D)))
    cbt = cb.T
    cbn = jnp.sum(cb * cb, axis=1).reshape(1, _K)
    pt = jnp.pad(phrase_table, ((0, _PP - _P), (0, _DP - _D)))
    wd = jnp.pad(W_dec, ((0, _DP - _D), (0, 0)))
    bd = b_dec.reshape(1, _DIN)
    pn = position_number.astype(jnp.int32).reshape(_G, 1, _TB)

    full = lambda shape: pl.BlockSpec(shape, lambda i: (0, 0))
    out, z, zq = pl.pallas_call(
        _fused,
        grid=(_G,),
        in_specs=[
            pl.BlockSpec((1, 1, _TB), lambda i: (i, 0, 0)),
            pl.BlockSpec((_TB, _DIN), lambda i: (i, 0)),
            pl.BlockSpec((_TB, _DIN), lambda i: (i, 0)),
            full((_DIN, _DP)),
            full((1, _DP)),
            full((_K, _DP)),
            full((_DP, _K)),
            full((1, _K)),
            full((_PP, _DP)),
            full((_DP, _DIN)),
            full((1, _DIN)),
        ],
        out_specs=[
            pl.BlockSpec((_TB, _DIN), lambda i: (i, 0)),
            pl.BlockSpec((_TB, _D), lambda i: (i, 0)),
            pl.BlockSpec((_TB, _D), lambda i: (i, 0)),
        ],
        out_shape=[
            jax.ShapeDtypeStruct((_B, _DIN), jnp.float32),
            jax.ShapeDtypeStruct((_B, _D), jnp.float32),
            jax.ShapeDtypeStruct((_B, _D), jnp.float32),
        ],
        compiler_params=pltpu.CompilerParams(
            dimension_semantics=("parallel",)),
    )(pn, xt16, xp16, wm, bm, cb, cbt, cbn, pt, wd, bd)
    return (out, z, zq)


# speedup vs baseline: 1.1917x; 1.1917x over previous
import jax
import jax.numpy as jnp
from jax.experimental import pallas as pl
from jax.experimental.pallas import tpu as pltpu

_B, _DIN, _D, _K, _P = 4096, 1024, 510, 128, 332
_DP = 512
_PP = 384
_TB = 256
_G = _B // _TB


def _fused(pn_ref, xt_ref, xp_ref, wm_ref, bm_ref, cb_ref, cbt_ref, cbn_ref,
           pt_ref, wd_ref, bd_ref, out_ref, z_ref, zq_ref):
    wm = wm_ref[...]
    bm = bm_ref[...]
    hi = jax.lax.Precision.HIGHEST
    zt = jnp.dot(xt_ref[...], wm, preferred_element_type=jnp.float32) + bm
    zp = jnp.dot(xp_ref[...], wm, preferred_element_type=jnp.float32) + bm

    cb = cb_ref[...]
    cbt = cbt_ref[...]
    cn = cbn_ref[...]
    st = cn - 2.0 * jnp.dot(
        zt, cbt, preferred_element_type=jnp.float32, precision=hi)
    sp = cn - 2.0 * jnp.dot(
        zp, cbt, preferred_element_type=jnp.float32, precision=hi)
    kt = jnp.argmin(st, axis=1)
    kp = jnp.argmin(sp, axis=1)
    iota_k = jax.lax.broadcasted_iota(jnp.int32, (_TB, _K), 1)
    oh_t = (iota_k == kt[:, None]).astype(jnp.float32)
    oh_p = (iota_k == kp[:, None]).astype(jnp.float32)
    zqt = jnp.dot(oh_t, cb, preferred_element_type=jnp.float32, precision=hi)
    zqp = jnp.dot(oh_p, cb, preferred_element_type=jnp.float32, precision=hi)

    pn = pn_ref[0, 0, :]
    iota_p = jax.lax.broadcasted_iota(jnp.int32, (_TB, _PP), 1)
    oh_pos = (iota_p == pn[:, None]).astype(jnp.float32)
    pos = jnp.dot(oh_pos, pt_ref[...], preferred_element_type=jnp.float32,
                  precision=hi)

    acc = zqt + zqp + pos
    logits = jnp.dot(acc, wd_ref[...],
                     preferred_element_type=jnp.float32) + bd_ref[...]
    out_ref[...] = jax.nn.sigmoid(logits)
    z_ref[...] = zt[:, :_D]
    zq_ref[...] = zqt[:, :_D]


def kernel(train_data, pre_phrase, position_number, W_mean, b_mean, W_var,
           b_var, W_dec, b_dec, codebook, phrase_table):
    del W_var, b_var

    wm = jnp.pad(W_mean, ((0, 0), (0, _DP - _D)))
    bm = jnp.pad(b_mean, (0, _DP - _D)).reshape(1, _DP)
    cb = jnp.pad(codebook, ((0, 0), (0, _DP - _D)))
    cbt = cb.T
    cbn = jnp.sum(cb * cb, axis=1).reshape(1, _K)
    pt = jnp.pad(phrase_table, ((0, _PP - _P), (0, _DP - _D)))
    wd = jnp.pad(W_dec, ((0, _DP - _D), (0, 0)))
    bd = b_dec.reshape(1, _DIN)
    pn = position_number.astype(jnp.int32).reshape(_G, 1, _TB)

    full = lambda shape: pl.BlockSpec(shape, lambda i: (0, 0))
    out, z, zq = pl.pallas_call(
        _fused,
        grid=(_G,),
        in_specs=[
            pl.BlockSpec((1, 1, _TB), lambda i: (i, 0, 0)),
            pl.BlockSpec((_TB, _DIN), lambda i: (i, 0)),
            pl.BlockSpec((_TB, _DIN), lambda i: (i, 0)),
            full((_DIN, _DP)),
            full((1, _DP)),
            full((_K, _DP)),
            full((_DP, _K)),
            full((1, _K)),
            full((_PP, _DP)),
            full((_DP, _DIN)),
            full((1, _DIN)),
        ],
        out_specs=[
            pl.BlockSpec((_TB, _DIN), lambda i: (i, 0)),
            pl.BlockSpec((_TB, _D), lambda i: (i, 0)),
            pl.BlockSpec((_TB, _D), lambda i: (i, 0)),
        ],
        out_shape=[
            jax.ShapeDtypeStruct((_B, _DIN), jnp.float32),
            jax.ShapeDtypeStruct((_B, _D), jnp.float32),
            jax.ShapeDtypeStruct((_B, _D), jnp.float32),
        ],
        compiler_params=pltpu.CompilerParams(
            dimension_semantics=("parallel",)),
    )(pn, train_data, pre_phrase, wm, bm, cb, cbt, cbn, pt, wd, bd)
    return (out, z, zq)


# speedup vs baseline: 2.0870x; 1.7513x over previous
import jax
import jax.numpy as jnp
from jax.experimental import pallas as pl
from jax.experimental.pallas import tpu as pltpu

_B, _DIN, _D, _K, _P = 4096, 1024, 510, 128, 332
_DP = 512
_PP = 384
_TB = 1024
_G = _B // _TB


def _fused(pn_ref, xt_ref, xp_ref, wm_ref, bm_ref, cb_ref, pt_ref, wd_ref,
           bd_ref, out_ref, z_ref, zq_ref):
    wm = wm_ref[...]
    bm = bm_ref[...]
    hi = jax.lax.Precision.HIGHEST
    zt = jnp.dot(xt_ref[...], wm, preferred_element_type=jnp.float32) + bm
    zp = jnp.dot(xp_ref[...], wm, preferred_element_type=jnp.float32) + bm

    cbf = cb_ref[...]
    cb = cbf.astype(jnp.bfloat16)
    dims = (((1,), (1,)), ((), ()))
    ones = jnp.ones((1, _D), jnp.float32)
    cn = jax.lax.dot_general(ones, cbf * cbf, dims,
                             preferred_element_type=jnp.float32,
                             precision=hi)
    st = cn - 2.0 * jax.lax.dot_general(
        zt, cbf, dims, preferred_element_type=jnp.float32, precision=hi)
    sp = cn - 2.0 * jax.lax.dot_general(
        zp, cbf, dims, preferred_element_type=jnp.float32, precision=hi)
    kt = jnp.argmin(st, axis=1)
    kp = jnp.argmin(sp, axis=1)
    iota_k = jax.lax.broadcasted_iota(jnp.int32, (_TB, _K), 1)
    oh_t = (iota_k == kt[:, None]).astype(jnp.bfloat16)
    oh_p = (iota_k == kp[:, None]).astype(jnp.bfloat16)
    zqt = jnp.dot(oh_t, cb, preferred_element_type=jnp.float32)
    zqp = jnp.dot(oh_p, cb, preferred_element_type=jnp.float32)

    pn = pn_ref[0, 0, :]
    iota_p = jax.lax.broadcasted_iota(jnp.int32, (_TB, _PP), 1)
    oh_pos = (iota_p == pn[:, None]).astype(jnp.bfloat16)
    pos = jnp.dot(oh_pos, pt_ref[...].astype(jnp.bfloat16),
                  preferred_element_type=jnp.float32)

    acc = zqt + zqp + pos
    logits = jnp.dot(acc, wd_ref[...],
                     preferred_element_type=jnp.float32) + bd_ref[...]
    out_ref[...] = jax.nn.sigmoid(logits)
    z_ref[...] = zt
    zq_ref[...] = zqt


def kernel(train_data, pre_phrase, position_number, W_mean, b_mean, W_var,
           b_var, W_dec, b_dec, codebook, phrase_table):
    del W_var, b_var

    wm = W_mean
    bm = b_mean.reshape(1, _D)
    pt = jnp.pad(phrase_table, ((0, _PP - _P), (0, 0)))
    wd = W_dec
    bd = b_dec.reshape(1, _DIN)
    pn = position_number.astype(jnp.int32).reshape(_G, 1, _TB)

    full = lambda shape: pl.BlockSpec(shape, lambda i: (0, 0))
    out, z, zq = pl.pallas_call(
        _fused,
        grid=(_G,),
        in_specs=[
            pl.BlockSpec((1, 1, _TB), lambda i: (i, 0, 0)),
            pl.BlockSpec((_TB, _DIN), lambda i: (i, 0)),
            pl.BlockSpec((_TB, _DIN), lambda i: (i, 0)),
            full((_DIN, _D)),
            full((1, _D)),
            full((_K, _D)),
            full((_PP, _D)),
            full((_D, _DIN)),
            full((1, _DIN)),
        ],
        out_specs=[
            pl.BlockSpec((_TB, _DIN), lambda i: (i, 0)),
            pl.BlockSpec((_TB, _D), lambda i: (i, 0)),
            pl.BlockSpec((_TB, _D), lambda i: (i, 0)),
        ],
        out_shape=[
            jax.ShapeDtypeStruct((_B, _DIN), jnp.float32),
            jax.ShapeDtypeStruct((_B, _D), jnp.float32),
            jax.ShapeDtypeStruct((_B, _D), jnp.float32),
        ],
        compiler_params=pltpu.CompilerParams(
            dimension_semantics=("parallel",)),
    )(pn, train_data, pre_phrase, wm, bm, codebook, pt, wd, bd)
    return (out, z, zq)
